# baseline (speedup 1.0000x reference)
import functools

import jax
import jax.numpy as jnp
from jax import lax
from jax.experimental import pallas as pl
from jax.experimental.pallas import tpu as pltpu
from jax.experimental.pallas import tpu_sc as plsc

_D = 512
_B = 16384

_info = plsc.get_sparse_core_info()
_NC = _info.num_cores
_NS = _info.num_subcores
_NW = _NC * _NS
_BPW = _B // _NW
_C = 64
_NCHUNK = _BPW // _C
_NB = 3

_mesh = plsc.VectorSubcoreMesh(core_axis_name="c", subcore_axis_name="s")


@functools.partial(
    pl.kernel,
    mesh=_mesh,
    out_type=jax.ShapeDtypeStruct((_B, _D), jnp.float32),
    scratch_types=[
        pltpu.VMEM((_BPW,), jnp.int32),
        pltpu.VMEM((_NB, _C, _D), jnp.float32),
        [pltpu.SemaphoreType.DMA] * _NB,
        [pltpu.SemaphoreType.DMA] * _NB,
    ],
)
def _gather_rows(table_hbm, idx_hbm, out_hbm, idx_v, rows_v, gsems, ssems):
    wid = lax.axis_index("s") * _NC + lax.axis_index("c")
    base = wid * _BPW
    pltpu.sync_copy(idx_hbm.at[pl.ds(base, _BPW)], idx_v)

    def gather(j, b):
        return pltpu.make_async_copy(
            table_hbm.at[idx_v.at[pl.ds(j * _C, _C)]], rows_v.at[b], gsems[b]
        )

    def writeback(j, b):
        return pltpu.make_async_copy(
            rows_v.at[b], out_hbm.at[pl.ds(base + j * _C, _C)], ssems[b]
        )

    _AHEAD = _NB - 1
    for j in range(_AHEAD):
        gather(j, j % _NB).start()
    for j in range(_NCHUNK):
        b = j % _NB
        gather(j, b).wait()
        writeback(j, b).start()
        nxt = j + _AHEAD
        if nxt < _NCHUNK:
            if j >= 1:
                writeback(j - 1, (j - 1) % _NB).wait()
            gather(nxt, nxt % _NB).start()
    for j in range(_NCHUNK - _AHEAD - 1, _NCHUNK):
        writeback(j, j % _NB).wait()


def kernel(text_cache, prompt_ids):
    return _gather_rows(text_cache, prompt_ids.astype(jnp.int32))


# speedup vs baseline: 1.0290x; 1.0290x over previous
import functools

import jax
import jax.numpy as jnp
from jax import lax
from jax.experimental import pallas as pl
from jax.experimental.pallas import tpu as pltpu
from jax.experimental.pallas import tpu_sc as plsc

_D = 512
_B = 16384

_info = plsc.get_sparse_core_info()
_NC = _info.num_cores
_NS = _info.num_subcores
_NW = _NC * _NS
_BPW = _B // _NW
_C = 120
_SIZES = [120, 120, 120, 120, 32]
_OFFS = [0, 120, 240, 360, 480]
_NCHUNK = len(_SIZES)
_NB = 2

_mesh = plsc.VectorSubcoreMesh(core_axis_name="c", subcore_axis_name="s")


@functools.partial(
    pl.kernel,
    mesh=_mesh,
    out_type=jax.ShapeDtypeStruct((_B, _D), jnp.float32),
    scratch_types=[
        pltpu.VMEM((_BPW,), jnp.int32),
        pltpu.VMEM((_NB, _C, _D), jnp.float32),
        [pltpu.SemaphoreType.DMA] * _NB,
        [pltpu.SemaphoreType.DMA] * _NB,
    ],
)
def _gather_rows(table_hbm, idx_hbm, out_hbm, idx_v, rows_v, gsems, ssems):
    wid = lax.axis_index("s") * _NC + lax.axis_index("c")
    base = wid * _BPW
    pltpu.sync_copy(idx_hbm.at[pl.ds(base, _BPW)], idx_v)

    def gather(j, b):
        n = _SIZES[j]
        return pltpu.make_async_copy(
            table_hbm.at[idx_v.at[pl.ds(_OFFS[j], n)]],
            rows_v.at[b, pl.ds(0, n)],
            gsems[b],
        )

    def writeback(j, b):
        n = _SIZES[j]
        return pltpu.make_async_copy(
            rows_v.at[b, pl.ds(0, n)],
            out_hbm.at[pl.ds(base + _OFFS[j], n)],
            ssems[b],
        )

    gather(0, 0).start()
    for j in range(_NCHUNK):
        b = j % _NB
        if j + 1 < _NCHUNK:
            gather(j + 1, (j + 1) % _NB).start()
        gather(j, b).wait()
        writeback(j, b).start()
        writeback(j, b).wait()


def kernel(text_cache, prompt_ids):
    return _gather_rows(text_cache, prompt_ids.astype(jnp.int32))
